# baseline (speedup 1.0000x reference)
import functools

import jax
import jax.numpy as jnp
from jax import lax
from jax.experimental import pallas as pl
from jax.experimental.pallas import tpu as pltpu
from jax.experimental.pallas import tpu_sc as plsc

N_NODES = 10000
D = 128
E = 320000
L = 16
NC = 2
NS = 16
NW = NC * NS
E_PER = 10240
E_PAD = NW * E_PER
K = 80
NCHUNK = E_PER // K
NB = 4
NI = 8
QN = NCHUNK // NI
N_ACC = 10240
ROWS_PER_TILE = N_ACC // NS
ZROWS = 128


def _mm_body(x_ref, w_ref, o_ref):
    o_ref[...] = jnp.dot(x_ref[...], w_ref[...], preferred_element_type=jnp.float32)


def _matmul(x, W):
    M = x.shape[0]
    BM = 1000
    return pl.pallas_call(
        _mm_body,
        grid=(M // BM,),
        in_specs=[
            pl.BlockSpec((BM, D), lambda i: (i, 0)),
            pl.BlockSpec((D, D), lambda i: (0, 0)),
        ],
        out_specs=pl.BlockSpec((BM, D), lambda i: (i, 0)),
        out_shape=jax.ShapeDtypeStruct((M, D), jnp.float32),
    )(x, W)


def _fin_body(p_ref, a_ref, o_ref):
    s = p_ref[0] + p_ref[1]
    a = a_ref[0]
    o_ref[...] = jnp.where(s >= 0.0, s, a * s)


def _finish(partials, prelu_a):
    BM = 1000
    return pl.pallas_call(
        _fin_body,
        grid=(N_NODES // BM,),
        in_specs=[
            pl.BlockSpec((NC, BM, D), lambda i: (0, i, 0)),
            pl.BlockSpec(memory_space=pltpu.SMEM),
        ],
        out_specs=pl.BlockSpec((BM, D), lambda i: (i, 0)),
        out_shape=jax.ShapeDtypeStruct((N_NODES, D), jnp.float32),
    )(partials, prelu_a)


def _lane_splat(v, lane):
    return lax.gather(
        v,
        jnp.full((L, 1), lane, jnp.int32),
        dimension_numbers=lax.GatherDimensionNumbers(
            offset_dims=(), collapsed_slice_dims=(0,), start_index_map=(0,)),
        slice_sizes=(1,),
        mode=lax.GatherScatterMode.PROMISE_IN_BOUNDS,
    )


def _sc_spmm(h, row1, col1, vals1):
    mesh = plsc.VectorSubcoreMesh(core_axis_name="c", subcore_axis_name="s")

    @functools.partial(
        pl.kernel,
        mesh=mesh,
        out_type=jax.ShapeDtypeStruct((NC, N_ACC, D), jnp.float32),
        scratch_types=[
            pltpu.VMEM((NI, K), jnp.int32),
            pltpu.VMEM((NI, K), jnp.int32),
            pltpu.VMEM((NI, K), jnp.float32),
            pltpu.VMEM((K, D), jnp.float32),
            pltpu.VMEM((K, D), jnp.float32),
            pltpu.VMEM((K, D), jnp.float32),
            pltpu.VMEM((K, D), jnp.float32),
            pltpu.SemaphoreType.DMA,
            pltpu.SemaphoreType.DMA,
            pltpu.SemaphoreType.DMA,
            pltpu.SemaphoreType.DMA,
            pltpu.SemaphoreType.DMA,
            pltpu.SemaphoreType.DMA,
            pltpu.SemaphoreType.DMA,
            pltpu.SemaphoreType.DMA,
            pltpu.SemaphoreType.DMA,
            pltpu.SemaphoreType.DMA,
            pltpu.SemaphoreType.DMA,
            pltpu.SemaphoreType.DMA,
            pltpu.SemaphoreType.DMA,
            pltpu.SemaphoreType.DMA,
            pltpu.SemaphoreType.DMA,
            pltpu.SemaphoreType.DMA,
            pltpu.VMEM_SHARED((N_ACC, D), jnp.float32),
        ],
    )
    def spmm(h_hbm, row_hbm, col_hbm, vals_hbm, out_hbm,
             colr, rowr, valr, b0, b1, b2, b3,
             i0, i1, i2, i3, i4, i5, i6, i7,
             g0, g1, g2, g3, s0, s1, s2, s3, acc):
        c = lax.axis_index("c")
        s = lax.axis_index("s")
        wid = c * NS + s
        ebase = wid * E_PER
        bufs = (b0, b1, b2, b3)
        isems = (i0, i1, i2, i3, i4, i5, i6, i7)
        gsems = (g0, g1, g2, g3)
        ssems = (s0, s1, s2, s3)

        zero16 = jnp.zeros((L,), jnp.float32)

        def zrow(i, carry):
            for j in range(D // L):
                b0[i, pl.ds(j * L, L)] = zero16
            return carry

        lax.fori_loop(0, K, zrow, 0)
        for z in range(ROWS_PER_TILE // K):
            pltpu.sync_copy(b0, acc.at[pl.ds(s * ROWS_PER_TILE + z * K, K)])
        plsc.subcore_barrier()

        def idx_start(ci, o):
            pltpu.async_copy(col_hbm.at[pl.ds(ebase + ci * K, K)],
                             colr.at[o], isems[o])
            pltpu.async_copy(row_hbm.at[pl.ds(ebase + ci * K, K)],
                             rowr.at[o], isems[o])
            pltpu.async_copy(vals_hbm.at[pl.ds(ebase + ci * K, K)],
                             valr.at[o], isems[o])

        def idx_wait(ci, o):
            pltpu.make_async_copy(col_hbm.at[pl.ds(ebase + ci * K, K)],
                                  colr.at[o], isems[o]).wait()
            pltpu.make_async_copy(row_hbm.at[pl.ds(ebase + ci * K, K)],
                                  rowr.at[o], isems[o]).wait()
            pltpu.make_async_copy(vals_hbm.at[pl.ds(ebase + ci * K, K)],
                                  valr.at[o], isems[o]).wait()

        def g_start(o, b):
            pltpu.async_copy(h_hbm.at[colr.at[o]], bufs[b], gsems[b])

        def g_wait(o, b):
            pltpu.make_async_copy(h_hbm.at[colr.at[o]], bufs[b],
                                  gsems[b]).wait()

        def scat_start(o, b):
            pltpu.async_copy(bufs[b], acc.at[rowr.at[o]], ssems[b], add=True)

        def scat_wait(o, b):
            pltpu.make_async_copy(bufs[b], acc.at[rowr.at[o]],
                                  ssems[b]).wait()

        def scale(o, b):
            buf = bufs[b]

            def grp(g, carry):
                vv = valr[o, pl.ds(g * L, L)]
                for lane in range(L):
                    sp = _lane_splat(vv, lane)
                    e = g * L + lane
                    for j in range(D // L):
                        buf[e, pl.ds(j * L, L)] = buf[e, pl.ds(j * L, L)] * sp
                return carry

            lax.fori_loop(0, K // L, grp, 0)

        for i in range(NB):
            idx_start(i, i)
        for i in range(2):
            idx_wait(i, i)
            g_start(i, i)

        def oct_(q, carry):
            for u in range(NI):
                b = u % NB

                if u < NB:
                    idx_start(NI * q + u + NB, (u + NB) % NI)
                else:

                    @pl.when(q < QN - 1)
                    def _():
                        idx_start(NI * q + u + NB, (u + NB) % NI)

                g_wait(u, b)
                scale(u, b)
                scat_start(u, b)

                pb = (b + 2) % NB
                po = (u + 2) % NI
                if u < 2:

                    @pl.when(q > 0)
                    def _():
                        scat_wait(po, pb)

                    idx_wait(NI * q + u + 2, po)
                    g_start(po, pb)
                elif u < NI - 2:
                    scat_wait(po, pb)
                    idx_wait(NI * q + u + 2, po)
                    g_start(po, pb)
                else:

                    @pl.when(q < QN - 1)
                    def _():
                        scat_wait(po, pb)
                        idx_wait(NI * q + u + 2, po)
                        g_start(po, pb)
            return carry

        lax.fori_loop(0, QN, oct_, 0)
        for t in range(NB):
            ci = NCHUNK - NB + t
            scat_wait(ci % NI, ci % NB)

        plsc.subcore_barrier()
        for z in range(ROWS_PER_TILE // ZROWS):
            base = s * ROWS_PER_TILE + z * ZROWS
            pltpu.sync_copy(acc.at[pl.ds(base, ZROWS)],
                            out_hbm.at[c, pl.ds(base, ZROWS)])

    return spmm(h, row1, col1, vals1)


def kernel(x, edge_index, adj_vals, W, prelu_a):
    h = _matmul(x, W)
    row = edge_index[0].astype(jnp.int32)
    col = edge_index[1].astype(jnp.int32)
    pad = E_PAD - E
    row1 = jnp.pad(row, (0, pad))
    col1 = jnp.pad(col, (0, pad))
    vals1 = jnp.pad(adj_vals, (0, pad))
    partials = _sc_spmm(h, row1, col1, vals1)[:, :N_NODES]
    a = jnp.reshape(prelu_a, (1,)).astype(jnp.float32)
    return _finish(partials, a)


# speedup vs baseline: 1.2836x; 1.2836x over previous
import functools

import jax
import jax.numpy as jnp
from jax import lax
from jax.experimental import pallas as pl
from jax.experimental.pallas import tpu as pltpu
from jax.experimental.pallas import tpu_sc as plsc

N_NODES = 10000
D = 128
DW = D // 2
E = 320000
L = 16
NC = 2
NS = 16
NW = NC * NS
E_PER = 10080
E_PAD = NW * E_PER
K = 80
KH = K // 2
NCHUNK = E_PER // K
PAIRS = NCHUNK // 2
N_ACC = 10240
ROWS_PER_TILE = N_ACC // NS
ZROWS = 128
HIMASK = jnp.int32(-65536)


def _mm_body(x_ref, w_ref, o_ref):
    o_ref[...] = jnp.dot(x_ref[...], w_ref[...],
                         preferred_element_type=jnp.float32).astype(jnp.bfloat16)


def _matmul(x, W):
    M = x.shape[0]
    BM = 1000
    return pl.pallas_call(
        _mm_body,
        grid=(M // BM,),
        in_specs=[
            pl.BlockSpec((BM, D), lambda i: (i, 0)),
            pl.BlockSpec((D, D), lambda i: (0, 0)),
        ],
        out_specs=pl.BlockSpec((BM, D), lambda i: (i, 0)),
        out_shape=jax.ShapeDtypeStruct((M, D), jnp.bfloat16),
    )(x, W)


def _fin_body(p_ref, a_ref, o_ref):
    s = p_ref[0] + p_ref[1]
    a = a_ref[0]
    o_ref[...] = jnp.where(s >= 0.0, s, a * s)


def _finish(partials, prelu_a):
    BM = 1000
    return pl.pallas_call(
        _fin_body,
        grid=(N_NODES // BM,),
        in_specs=[
            pl.BlockSpec((NC, BM, D), lambda i: (0, i, 0)),
            pl.BlockSpec(memory_space=pltpu.SMEM),
        ],
        out_specs=pl.BlockSpec((BM, D), lambda i: (i, 0)),
        out_shape=jax.ShapeDtypeStruct((N_NODES, D), jnp.float32),
    )(partials, prelu_a)


def _lane_splat(v, lane):
    return lax.gather(
        v,
        jnp.full((L, 1), lane, jnp.int32),
        dimension_numbers=lax.GatherDimensionNumbers(
            offset_dims=(), collapsed_slice_dims=(0,), start_index_map=(0,)),
        slice_sizes=(1,),
        mode=lax.GatherScatterMode.PROMISE_IN_BOUNDS,
    )


def _sc_spmm(hb, row3, col1, vals1):
    mesh = plsc.VectorSubcoreMesh(core_axis_name="c", subcore_axis_name="s")

    @functools.partial(
        pl.kernel,
        mesh=mesh,
        compiler_params=pltpu.CompilerParams(use_tc_tiling_on_sc=False,
                                             needs_layout_passes=False),
        out_type=jax.ShapeDtypeStruct((NC, N_ACC, D), jnp.float32),
        scratch_types=[
            pltpu.VMEM((E_PER,), jnp.int32),
            pltpu.VMEM((2 * NCHUNK, KH), jnp.int32),
            pltpu.VMEM((K + L,), jnp.float32),
            pltpu.VMEM((K + L,), jnp.float32),
            pltpu.VMEM((K, DW), jnp.int32),
            pltpu.VMEM((K, DW), jnp.int32),
            pltpu.VMEM((KH, D), jnp.float32),
            pltpu.VMEM((KH, D), jnp.float32),
            pltpu.VMEM_SHARED((N_ACC, D), jnp.float32),
            pltpu.SemaphoreType.DMA,
            pltpu.SemaphoreType.DMA,
            pltpu.SemaphoreType.DMA,
            pltpu.SemaphoreType.DMA,
        ],
    )
    def spmm(h_hbm, row_hbm, col_hbm, vals_hbm, out_hbm,
             col_v, row2_v, vbuf0, vbuf1, gb0, gb1, sb0, sb1, acc,
             gs0, gs1, ss0, ss1):
        c = lax.axis_index("c")
        s = lax.axis_index("s")
        wid = c * NS + s
        ebase = wid * E_PER
        vbufs = (vbuf0, vbuf1)
        gbufs = (gb0, gb1)
        sbufs = (sb0, sb1)
        ssems = (ss0, ss1)

        pltpu.sync_copy(col_hbm.at[pl.ds(ebase, E_PER)], col_v)
        pltpu.sync_copy(row_hbm.at[wid], row2_v)

        zero16 = jnp.zeros((L,), jnp.float32)

        def zrow(i, carry):
            for j in range(D // L):
                sb0[i, pl.ds(j * L, L)] = zero16
            return carry

        lax.fori_loop(0, KH, zrow, 0)
        for z in range(ROWS_PER_TILE // KH):
            pltpu.sync_copy(sb0, acc.at[pl.ds(s * ROWS_PER_TILE + z * KH, KH)])
        plsc.subcore_barrier()

        def gather_start(ci, b):
            pltpu.async_copy(h_hbm.at[col_v.at[pl.ds(ci * K, K)]],
                             gbufs[b], (gs0, gs1)[b])
            pltpu.async_copy(vals_hbm.at[pl.ds(ebase + ci * K, K)],
                             vbufs[b].at[pl.ds(0, K)], (gs0, gs1)[b])

        def gather_wait(ci, b):
            pltpu.make_async_copy(h_hbm.at[col_v.at[pl.ds(ci * K, K)]],
                                  gbufs[b], (gs0, gs1)[b]).wait()
            pltpu.make_async_copy(vals_hbm.at[pl.ds(ebase + ci * K, K)],
                                  vbufs[b].at[pl.ds(0, K)], (gs0, gs1)[b]).wait()

        def scat_start(ci, half, hb_):
            pltpu.async_copy(sbufs[hb_], acc.at[row2_v.at[2 * ci + half]],
                             ssems[hb_], add=True)

        def scat_wait(ci, half, hb_):
            pltpu.make_async_copy(sbufs[hb_], acc.at[row2_v.at[2 * ci + half]],
                                  ssems[hb_]).wait()

        def unpack_scale(gbuf, vbuf, sbuf, base, n, goff):
            def one_group(g, eoff):
                vv = vbuf[pl.ds(base, L)]
                for lane in range(n):
                    sp = _lane_splat(vv, lane)
                    e = base + lane
                    r = eoff + lane
                    for j in range(D // 32):
                        w = gbuf[e, pl.ds(j * L, L)]
                        flo = plsc.bitcast(w << 16, jnp.float32)
                        fhi = plsc.bitcast(w & HIMASK, jnp.float32)
                        sbuf[r, pl.ds(j * 32, L)] = flo * sp
                        sbuf[r, pl.ds(j * 32 + L, L)] = fhi * sp

            one_group(0, goff * L)

        def process(ci, b, has_prev):
            gbuf = gbufs[b]
            vbuf = vbufs[b]
            for half in range(2):
                sbuf = sbufs[half]
                if has_prev:
                    scat_wait(ci - 1, half, half)
                else:

                    @pl.when(ci > 0)
                    def _():
                        scat_wait(ci - 1, half, half)

                hb_base = half * KH

                def grp16(g, carry):
                    base = hb_base + g * L
                    vv = vbuf[pl.ds(base, L)]
                    for lane in range(L):
                        sp = _lane_splat(vv, lane)
                        e = base + lane
                        r = g * L + lane
                        for j in range(D // 32):
                            w = gbuf[e, pl.ds(j * L, L)]
                            flo = plsc.bitcast(w << 16, jnp.float32)
                            fhi = plsc.bitcast(w & HIMASK, jnp.float32)
                            sbuf[r, pl.ds(j * 32, L)] = flo * sp
                            sbuf[r, pl.ds(j * 32 + L, L)] = fhi * sp
                    return carry

                lax.fori_loop(0, 2, grp16, 0)
                base = hb_base + 2 * L
                vv = vbuf[pl.ds(base, L)]
                for lane in range(8):
                    sp = _lane_splat(vv, lane)
                    e = base + lane
                    r = 2 * L + lane
                    for j in range(D // 32):
                        w = gbuf[e, pl.ds(j * L, L)]
                        flo = plsc.bitcast(w << 16, jnp.float32)
                        fhi = plsc.bitcast(w & HIMASK, jnp.float32)
                        sbuf[r, pl.ds(j * 32, L)] = flo * sp
                        sbuf[r, pl.ds(j * 32 + L, L)] = fhi * sp
                scat_start(ci, half, half)

        gather_start(0, 0)

        def pair(p, carry):
            c0 = 2 * p
            gather_start(c0 + 1, 1)
            gather_wait(c0, 0)
            process(c0, 0, False)

            @pl.when(p < PAIRS - 1)
            def _():
                gather_start(c0 + 2, 0)

            gather_wait(c0 + 1, 1)
            process(c0 + 1, 1, True)
            return carry

        lax.fori_loop(0, PAIRS, pair, 0)
        scat_wait(NCHUNK - 1, 0, 0)
        scat_wait(NCHUNK - 1, 1, 1)

        plsc.subcore_barrier()
        for z in range(ROWS_PER_TILE // ZROWS):
            base = s * ROWS_PER_TILE + z * ZROWS
            pltpu.sync_copy(acc.at[pl.ds(base, ZROWS)],
                            out_hbm.at[c, pl.ds(base, ZROWS)])

    return spmm(hb, row3, col1, vals1)


def kernel(x, edge_index, adj_vals, W, prelu_a):
    Ws = W.reshape(D, 4, 2, L).transpose(0, 1, 3, 2).reshape(D, D)
    h = _matmul(x, Ws)
    hb = lax.bitcast_convert_type(h.reshape(N_NODES, DW, 2), jnp.int32)
    row = edge_index[0].astype(jnp.int32)
    col = edge_index[1].astype(jnp.int32)
    pad = E_PAD - E
    row3 = jnp.pad(row, (0, pad)).reshape(NW, 2 * NCHUNK, KH)
    col1 = jnp.pad(col, (0, pad))
    vals1 = jnp.pad(adj_vals, (0, pad))
    partials = _sc_spmm(hb, row3, col1, vals1)[:, :N_NODES]
    a = jnp.reshape(prelu_a, (1,)).astype(jnp.float32)
    return _finish(partials, a)
